# baseline (speedup 1.0000x reference)
import functools

import jax
import jax.numpy as jnp
from jax import lax
from jax.experimental import pallas as pl
from jax.experimental.pallas import tpu as pltpu
from jax.experimental.pallas import tpu_sc as plsc

_IDX_CHUNK = 128


@functools.cache
def _gather_fn(B, V, A):
    info = plsc.get_sparse_core_info()
    nw = info.num_cores * info.num_subcores
    b_per_w = B // nw
    n_ch = b_per_w // _IDX_CHUNK
    mesh = plsc.VectorSubcoreMesh(core_axis_name="c", subcore_axis_name="s")

    @functools.partial(
        pl.kernel,
        out_type=jax.ShapeDtypeStruct((B, A), jnp.float32),
        mesh=mesh,
        compiler_params=pltpu.CompilerParams(use_tc_tiling_on_sc=False),
        scratch_types=[
            pltpu.VMEM((n_ch, _IDX_CHUNK), jnp.int32),
            pltpu.VMEM((b_per_w, A), jnp.float32),
            pltpu.SemaphoreType.DMA,
        ],
    )
    def gather(idx_hbm, table_hbm, out_hbm, idx_v, rows_v, sem):
        wid = lax.axis_index("s") * info.num_cores + lax.axis_index("c")
        base = wid * b_per_w
        pltpu.sync_copy(idx_hbm.at[wid], idx_v)
        copies = [
            pltpu.async_copy(
                table_hbm.at[idx_v.at[j]],
                rows_v.at[pl.ds(j * _IDX_CHUNK, _IDX_CHUNK)],
                sem,
            )
            for j in range(n_ch)
        ]
        for c in copies:
            c.wait()
        pltpu.sync_copy(rows_v, out_hbm.at[pl.ds(base, b_per_w)])

    return gather


def _sample_body(p_ref, g_ref, act_ref):
    z = jnp.log(p_ref[...]) + g_ref[...]
    m = jnp.max(z, axis=1, keepdims=True)
    ii = lax.broadcasted_iota(jnp.int32, z.shape, 1)
    act_ref[...] = jnp.min(jnp.where(z == m, ii, z.shape[1]), axis=1)


@functools.cache
def _sample_fn(B, A, blk):
    grid = B // blk
    return pl.pallas_call(
        _sample_body,
        grid=(grid,),
        in_specs=[
            pl.BlockSpec((blk, A), lambda i: (i, 0)),
            pl.BlockSpec((blk, A), lambda i: (i, 0)),
        ],
        out_specs=pl.BlockSpec((blk,), lambda i: (i,)),
        out_shape=jax.ShapeDtypeStruct((B,), jnp.int32),
    )


def kernel(state, model):
    B = state.shape[0]
    V, A = model.shape
    info = plsc.get_sparse_core_info()
    nw = info.num_cores * info.num_subcores
    b_per_w = B // nw
    g = jax.random.gumbel(jax.random.key(42), (B, A), jnp.float32)
    idx = state.reshape(nw, b_per_w // _IDX_CHUNK, _IDX_CHUNK)
    action_probs = _gather_fn(B, V, A)(idx, model)
    actions = _sample_fn(B, A, 2048)(action_probs, g)
    return actions, action_probs
